# baseline (speedup 1.0000x reference)
import jax
import jax.numpy as jnp
from jax import lax
from jax.experimental import pallas as pl
from jax.experimental.pallas import tpu as pltpu
from jax.experimental.pallas import tpu_sc as plsc

N = 10000
E = 320000
K = 128
NC = 2
NS = 16
NW = NC * NS
L = 16
C = 80
PER_W = E // NW
NCHUNK = PER_W // C
SUP = 5
NSUPER = NCHUNK // SUP
NP = 10240
RPS = NP // NS


def _proj_body(att_ref, val_ref, ent_ref, aw_ref, ab_ref, w_ref,
               a_ref, v_ref, p_ref):
    w1 = w_ref[0:K, :]
    w2 = w_ref[K:2 * K, :]
    a_ref[...] = jnp.dot(att_ref[...], w1, preferred_element_type=jnp.float32)
    v_ref[...] = jnp.dot(val_ref[...], w2, preferred_element_type=jnp.float32)
    aw1 = aw_ref[0, 0:K]
    aw2 = aw_ref[0, K:2 * K]
    se = jnp.sum(ent_ref[...] * aw1[None, :], axis=1, keepdims=True)
    se = se + ab_ref[0, 0]
    sa = jnp.sum(att_ref[...] * aw2[None, :], axis=1, keepdims=True)
    p_ref[...] = jnp.concatenate(
        (se, sa, jnp.zeros((N, L - 2), jnp.float32)), axis=1)


def _project(att_feats, val_feats, ent_feats, a_w, a_b, w):
    return pl.pallas_call(
        _proj_body,
        out_shape=(
            jax.ShapeDtypeStruct((N, K), jnp.float32),
            jax.ShapeDtypeStruct((N, K), jnp.float32),
            jax.ShapeDtypeStruct((N, L), jnp.float32),
        ),
    )(att_feats, val_feats, ent_feats, a_w, a_b.reshape(1, 1), w)


def _sc_body(h_hbm, att_hbm, val_hbm, a_hbm, v_hbm, p_hbm,
             out_hbm, rs_hbm,
             hflat, aflat, vflat, hv5, arows, vrows, obuf, srows, ph, pa,
             out_sh, rs_sh, sem_g, sem_p, sem_s):
    cid = lax.axis_index("c")
    sid = lax.axis_index("s")
    wid = sid * NC + cid

    zeros_f = jnp.zeros((L,), jnp.float32)
    zeros_i = jnp.zeros((L,), jnp.int32)

    @pl.loop(0, C)
    def _(r):
        for q in range(K // L):
            obuf[r, pl.ds(q * L, L)] = zeros_f
        srows[r, :] = zeros_f

    for k in range(SUP):
        for g in range(C // L):
            hv5[k, pl.ds(g * L, L)] = zeros_i

    row0 = sid * RPS
    for i in range(RPS // C):
        pltpu.sync_copy(obuf, out_sh.at[pl.ds(row0 + i * C, C)])
        pltpu.sync_copy(srows, rs_sh.at[pl.ds(row0 + i * C, C)])
    plsc.subcore_barrier()

    base0 = wid * PER_W
    col0 = jnp.zeros((L,), jnp.int32)
    col1 = jnp.full((L,), 1, jnp.int32)

    pltpu.async_copy(obuf, out_sh.at[hv5.at[0]], sem_s, add=True)
    pltpu.async_copy(srows, rs_sh.at[hv5.at[0]], sem_s, add=True)

    @pl.loop(0, NSUPER)
    def _(s):
        sbase = base0 + s * SUP * C
        pltpu.sync_copy(h_hbm.at[pl.ds(sbase, SUP * C)], hflat)
        pltpu.sync_copy(att_hbm.at[pl.ds(sbase, SUP * C)], aflat)
        pltpu.sync_copy(val_hbm.at[pl.ds(sbase, SUP * C)], vflat)

        for k in range(SUP):
            off = k * C
            hk = hflat.at[pl.ds(off, C)]
            ak = aflat.at[pl.ds(off, C)]
            vk = vflat.at[pl.ds(off, C)]
            gp = pltpu.async_copy(p_hbm.at[hk], ph, sem_p)
            gq = pltpu.async_copy(p_hbm.at[ak], pa, sem_p)
            ga = pltpu.async_copy(a_hbm.at[ak], arows, sem_g)
            gv = pltpu.async_copy(v_hbm.at[vk], vrows, sem_g)

            for g in range(C // L):
                sl = pl.ds(g * L, L)
                hv5[k, sl] = hflat[pl.ds(off + g * L, L)]

            pltpu.make_async_copy(obuf, out_sh.at[hv5.at[k]], sem_s).wait()
            pltpu.make_async_copy(srows, rs_sh.at[hv5.at[k]], sem_s).wait()

            gp.wait()
            gq.wait()
            for g in range(C // L):
                idx = lax.iota(jnp.int32, L) + g * L
                x = (plsc.load_gather(ph, [idx, col0])
                     + plsc.load_gather(pa, [idx, col1]))
                sc = jnp.exp(jnp.maximum(x, 0.2 * x))
                plsc.store_scatter(srows, [idx, col0], sc)

            ga.wait()
            gv.wait()

            @plsc.parallel_loop(0, C, 1, unroll=8)
            def _(e):
                sb = plsc.load_gather(
                    srows, [jnp.full((L,), e, jnp.int32), col0])
                for q in range(K // L):
                    sl = pl.ds(q * L, L)
                    obuf[e, sl] = (arows[e, sl] + vrows[e, sl]) * sb

            pltpu.async_copy(obuf, out_sh.at[hv5.at[k]], sem_s, add=True)
            pltpu.async_copy(srows, rs_sh.at[hv5.at[k]], sem_s, add=True)

    pltpu.make_async_copy(obuf, out_sh.at[hv5.at[0]], sem_s).wait()
    pltpu.make_async_copy(srows, rs_sh.at[hv5.at[0]], sem_s).wait()

    plsc.subcore_barrier()

    pltpu.sync_copy(out_sh.at[pl.ds(row0, RPS)],
                    out_hbm.at[cid].at[pl.ds(row0, RPS)])
    pltpu.sync_copy(rs_sh.at[pl.ds(row0, RPS)],
                    rs_hbm.at[cid].at[pl.ds(row0, RPS)])


def _sc_aggregate(h, att, val, a_tab, v_tab, p_tab):
    mesh = plsc.VectorSubcoreMesh(core_axis_name="c", subcore_axis_name="s")
    cp = pltpu.CompilerParams(needs_layout_passes=False,
                              use_tc_tiling_on_sc=False)
    kern = pl.kernel(
        _sc_body,
        compiler_params=cp,
        out_type=(
            jax.ShapeDtypeStruct((NC, NP, K), jnp.float32),
            jax.ShapeDtypeStruct((NC, NP, L), jnp.float32),
        ),
        mesh=mesh,
        scratch_types=[
            pltpu.VMEM((SUP * C,), jnp.int32),
            pltpu.VMEM((SUP * C,), jnp.int32),
            pltpu.VMEM((SUP * C,), jnp.int32),
            pltpu.VMEM((SUP, C), jnp.int32),
            pltpu.VMEM((C, K), jnp.float32),
            pltpu.VMEM((C, K), jnp.float32),
            pltpu.VMEM((C, K), jnp.float32),
            pltpu.VMEM((C, L), jnp.float32),
            pltpu.VMEM((C, L), jnp.float32),
            pltpu.VMEM((C, L), jnp.float32),
            pltpu.VMEM_SHARED((NP, K), jnp.float32),
            pltpu.VMEM_SHARED((NP, L), jnp.float32),
            pltpu.SemaphoreType.DMA,
            pltpu.SemaphoreType.DMA,
            pltpu.SemaphoreType.DMA,
        ],
    )
    return kern(h, att, val, a_tab, v_tab, p_tab)


def _combine_body(po_ref, prs_ref, ent_ref, out_ref):
    tot = po_ref[0, 0:N] + po_ref[1, 0:N]
    rs = prs_ref[0, 0:N, 0] + prs_ref[1, 0:N, 0]
    rs = jnp.where(rs == 0.0, 1.0, rs)
    t = tot / rs[:, None] + ent_ref[...]
    out_ref[...] = jnp.where(t > 0.0, t, jnp.exp(jnp.minimum(t, 0.0)) - 1.0)


def _combine(out_pair, rs_pair, ent_feats):
    return pl.pallas_call(
        _combine_body,
        out_shape=jax.ShapeDtypeStruct((N, K), jnp.float32),
    )(out_pair, rs_pair, ent_feats)


def kernel(attribute_triples, att_feats, val_feats, ent_feats, a_w, a_b, W):
    h = attribute_triples[:, 0]
    val = attribute_triples[:, 1]
    att = attribute_triples[:, 2]
    a_tab, v_tab, p_tab = _project(att_feats, val_feats, ent_feats,
                                   a_w, a_b, W)
    out_pair, rs_pair = _sc_aggregate(h, att, val, a_tab, v_tab, p_tab)
    return _combine(out_pair, rs_pair, ent_feats)


# speedup vs baseline: 1.0347x; 1.0347x over previous
import jax
import jax.numpy as jnp
from jax import lax
from jax.experimental import pallas as pl
from jax.experimental.pallas import tpu as pltpu
from jax.experimental.pallas import tpu_sc as plsc

N = 10000
E = 320000
K = 128
NC = 2
NS = 16
NW = NC * NS
L = 16
C = 80
PER_W = E // NW
NCHUNK = PER_W // C
SUP = 5
NSUPER = NCHUNK // SUP
NP = 10240
RPS = NP // NS


def _proj_body(att_ref, val_ref, ent_ref, aw_ref, ab_ref, w_ref,
               a_ref, v_ref, p_ref):
    w1 = w_ref[0:K, :]
    w2 = w_ref[K:2 * K, :]
    a_ref[...] = jnp.dot(att_ref[...], w1, preferred_element_type=jnp.float32)
    v_ref[...] = jnp.dot(val_ref[...], w2, preferred_element_type=jnp.float32)
    aw1 = aw_ref[0, 0:K]
    aw2 = aw_ref[0, K:2 * K]
    se = jnp.sum(ent_ref[...] * aw1[None, :], axis=1, keepdims=True)
    se = se + ab_ref[0, 0]
    sa = jnp.sum(att_ref[...] * aw2[None, :], axis=1, keepdims=True)
    p_ref[...] = jnp.concatenate(
        (se, sa, jnp.zeros((N, L - 2), jnp.float32)), axis=1)


def _project(att_feats, val_feats, ent_feats, a_w, a_b, w):
    return pl.pallas_call(
        _proj_body,
        out_shape=(
            jax.ShapeDtypeStruct((N, K), jnp.float32),
            jax.ShapeDtypeStruct((N, K), jnp.float32),
            jax.ShapeDtypeStruct((N, L), jnp.float32),
        ),
    )(att_feats, val_feats, ent_feats, a_w, a_b.reshape(1, 1), w)


def _sc_body(h_hbm, att_hbm, val_hbm, a_hbm, v_hbm, p_hbm,
             out_hbm, rs_hbm,
             hflat, aflat, vflat, hv5, arows, vrows, obuf, srows, ph, pa,
             out_sh, rs_sh, sem_g, sem_p, sem_s, sem_r):
    cid = lax.axis_index("c")
    sid = lax.axis_index("s")
    wid = sid * NC + cid

    zeros_f = jnp.zeros((L,), jnp.float32)
    zeros_i = jnp.zeros((L,), jnp.int32)

    @pl.loop(0, C)
    def _(r):
        for q in range(K // L):
            obuf[r, pl.ds(q * L, L)] = zeros_f
        srows[r, :] = zeros_f

    for k in range(SUP):
        for g in range(C // L):
            hv5[k, pl.ds(g * L, L)] = zeros_i

    row0 = sid * RPS
    for i in range(RPS // C):
        pltpu.sync_copy(obuf, out_sh.at[pl.ds(row0 + i * C, C)])
        pltpu.sync_copy(srows, rs_sh.at[pl.ds(row0 + i * C, C)])
    plsc.subcore_barrier()

    base0 = wid * PER_W
    col0 = jnp.zeros((L,), jnp.int32)
    col1 = jnp.full((L,), 1, jnp.int32)

    pltpu.async_copy(obuf, out_sh.at[hv5.at[0]], sem_s, add=True)
    pltpu.async_copy(srows, rs_sh.at[hv5.at[0]], sem_r, add=True)

    @pl.loop(0, NSUPER)
    def _(s):
        sbase = base0 + s * SUP * C
        pltpu.sync_copy(h_hbm.at[pl.ds(sbase, SUP * C)], hflat)
        pltpu.sync_copy(att_hbm.at[pl.ds(sbase, SUP * C)], aflat)
        pltpu.sync_copy(val_hbm.at[pl.ds(sbase, SUP * C)], vflat)

        for k in range(SUP):
            off = k * C
            hk = hflat.at[pl.ds(off, C)]
            ak = aflat.at[pl.ds(off, C)]
            vk = vflat.at[pl.ds(off, C)]
            gp = pltpu.async_copy(p_hbm.at[hk], ph, sem_p)
            gq = pltpu.async_copy(p_hbm.at[ak], pa, sem_p)
            ga = pltpu.async_copy(a_hbm.at[ak], arows, sem_g)
            gv = pltpu.async_copy(v_hbm.at[vk], vrows, sem_g)

            for g in range(C // L):
                sl = pl.ds(g * L, L)
                hv5[k, sl] = hflat[pl.ds(off + g * L, L)]

            pltpu.make_async_copy(srows, rs_sh.at[hv5.at[k]], sem_r).wait()

            gp.wait()
            gq.wait()
            for g in range(C // L):
                idx = lax.iota(jnp.int32, L) + g * L
                x = (plsc.load_gather(ph, [idx, col0])
                     + plsc.load_gather(pa, [idx, col1]))
                sc = jnp.exp(jnp.maximum(x, 0.2 * x))
                plsc.store_scatter(srows, [idx, col0], sc)

            ga.wait()
            gv.wait()
            pltpu.make_async_copy(obuf, out_sh.at[hv5.at[k]], sem_s).wait()

            @plsc.parallel_loop(0, C, 1, unroll=4)
            def _(e):
                sb = plsc.load_gather(
                    srows, [jnp.full((L,), e, jnp.int32), col0])
                for q in range(K // L):
                    sl = pl.ds(q * L, L)
                    obuf[e, sl] = (arows[e, sl] + vrows[e, sl]) * sb

            pltpu.async_copy(obuf, out_sh.at[hv5.at[k]], sem_s, add=True)
            pltpu.async_copy(srows, rs_sh.at[hv5.at[k]], sem_r, add=True)

    pltpu.make_async_copy(obuf, out_sh.at[hv5.at[0]], sem_s).wait()
    pltpu.make_async_copy(srows, rs_sh.at[hv5.at[0]], sem_r).wait()

    plsc.subcore_barrier()

    pltpu.sync_copy(out_sh.at[pl.ds(row0, RPS)],
                    out_hbm.at[cid].at[pl.ds(row0, RPS)])
    pltpu.sync_copy(rs_sh.at[pl.ds(row0, RPS)],
                    rs_hbm.at[cid].at[pl.ds(row0, RPS)])


def _sc_aggregate(h, att, val, a_tab, v_tab, p_tab):
    mesh = plsc.VectorSubcoreMesh(core_axis_name="c", subcore_axis_name="s")
    cp = pltpu.CompilerParams(needs_layout_passes=False,
                              use_tc_tiling_on_sc=False)
    kern = pl.kernel(
        _sc_body,
        compiler_params=cp,
        out_type=(
            jax.ShapeDtypeStruct((NC, NP, K), jnp.float32),
            jax.ShapeDtypeStruct((NC, NP, L), jnp.float32),
        ),
        mesh=mesh,
        scratch_types=[
            pltpu.VMEM((SUP * C,), jnp.int32),
            pltpu.VMEM((SUP * C,), jnp.int32),
            pltpu.VMEM((SUP * C,), jnp.int32),
            pltpu.VMEM((SUP, C), jnp.int32),
            pltpu.VMEM((C, K), jnp.float32),
            pltpu.VMEM((C, K), jnp.float32),
            pltpu.VMEM((C, K), jnp.float32),
            pltpu.VMEM((C, L), jnp.float32),
            pltpu.VMEM((C, L), jnp.float32),
            pltpu.VMEM((C, L), jnp.float32),
            pltpu.VMEM_SHARED((NP, K), jnp.float32),
            pltpu.VMEM_SHARED((NP, L), jnp.float32),
            pltpu.SemaphoreType.DMA,
            pltpu.SemaphoreType.DMA,
            pltpu.SemaphoreType.DMA,
            pltpu.SemaphoreType.DMA,
        ],
    )
    return kern(h, att, val, a_tab, v_tab, p_tab)


def _combine_body(po_ref, prs_ref, ent_ref, out_ref):
    tot = po_ref[0, 0:N] + po_ref[1, 0:N]
    rs = prs_ref[0, 0:N, 0] + prs_ref[1, 0:N, 0]
    rs = jnp.where(rs == 0.0, 1.0, rs)
    t = tot / rs[:, None] + ent_ref[...]
    out_ref[...] = jnp.where(t > 0.0, t, jnp.exp(jnp.minimum(t, 0.0)) - 1.0)


def _combine(out_pair, rs_pair, ent_feats):
    return pl.pallas_call(
        _combine_body,
        out_shape=jax.ShapeDtypeStruct((N, K), jnp.float32),
    )(out_pair, rs_pair, ent_feats)


def kernel(attribute_triples, att_feats, val_feats, ent_feats, a_w, a_b, W):
    h = attribute_triples[:, 0]
    val = attribute_triples[:, 1]
    att = attribute_triples[:, 2]
    a_tab, v_tab, p_tab = _project(att_feats, val_feats, ent_feats,
                                   a_w, a_b, W)
    out_pair, rs_pair = _sc_aggregate(h, att, val, a_tab, v_tab, p_tab)
    return _combine(out_pair, rs_pair, ent_feats)


# speedup vs baseline: 1.4405x; 1.3922x over previous
import jax
import jax.numpy as jnp
from jax import lax
from jax.experimental import pallas as pl
from jax.experimental.pallas import tpu as pltpu
from jax.experimental.pallas import tpu_sc as plsc

N = 10000
E = 320000
K = 128
NC = 2
NS = 16
NW = NC * NS
L = 16
CB = 40
HPS = 10
SUPE = CB * HPS
PER_W = E // NW
NSUPER = PER_W // SUPE
NP = 10240
RPS = NP // NS


def _proj_body(att_ref, val_ref, ent_ref, aw_ref, ab_ref, w_ref,
               a_ref, v_ref, p_ref):
    w1 = w_ref[0:K, :]
    w2 = w_ref[K:2 * K, :]
    a_ref[...] = jnp.dot(att_ref[...], w1, preferred_element_type=jnp.float32)
    v_ref[...] = jnp.dot(val_ref[...], w2, preferred_element_type=jnp.float32)
    aw1 = aw_ref[0, 0:K]
    aw2 = aw_ref[0, K:2 * K]
    se = jnp.sum(ent_ref[...] * aw1[None, :], axis=1, keepdims=True)
    se = se + ab_ref[0, 0]
    sa = jnp.sum(att_ref[...] * aw2[None, :], axis=1, keepdims=True)
    p_ref[...] = jnp.concatenate(
        (se, sa, jnp.zeros((N, L - 2), jnp.float32)), axis=1)


def _project(att_feats, val_feats, ent_feats, a_w, a_b, w):
    return pl.pallas_call(
        _proj_body,
        out_shape=(
            jax.ShapeDtypeStruct((N, K), jnp.float32),
            jax.ShapeDtypeStruct((N, K), jnp.float32),
            jax.ShapeDtypeStruct((N, L), jnp.float32),
        ),
    )(att_feats, val_feats, ent_feats, a_w, a_b.reshape(1, 1), w)


def _sc_body(h_hbm, att_hbm, val_hbm, a_hbm, v_hbm, p_hbm,
             out_hbm, rs_hbm,
             hflat2, aflat2, vflat2, hv, ar2, vr2, ob2, sr2, ph2, pa2,
             out_sh, rs_sh,
             sem_g0, sem_g1, sem_p0, sem_p1,
             sem_s0, sem_s1, sem_r0, sem_r1, sem_i):
    cid = lax.axis_index("c")
    sid = lax.axis_index("s")
    wid = sid * NC + cid

    sem_g = (sem_g0, sem_g1)
    sem_p = (sem_p0, sem_p1)
    sem_s = (sem_s0, sem_s1)
    sem_r = (sem_r0, sem_r1)

    zeros_f = jnp.zeros((L,), jnp.float32)
    zeros_i = jnp.zeros((L,), jnp.int32)

    @pl.loop(0, CB)
    def _(r):
        for bb in range(2):
            for q in range(K // L):
                ob2[bb, r, pl.ds(q * L, L)] = zeros_f
            sr2[bb, r, :] = zeros_f

    for kk in range(HPS):
        hv[kk, pl.ds(0, L)] = zeros_i
        hv[kk, pl.ds(L, L)] = zeros_i
        hv[kk, pl.ds(CB - L, L)] = zeros_i

    row0 = sid * RPS
    for i in range(RPS // CB):
        pltpu.async_copy(ob2.at[0], out_sh.at[pl.ds(row0 + i * CB, CB)],
                         sem_s0)
        pltpu.async_copy(sr2.at[0], rs_sh.at[pl.ds(row0 + i * CB, CB)],
                         sem_r0)
    for i in range(RPS // CB):
        pltpu.make_async_copy(ob2.at[0], out_sh.at[pl.ds(row0, CB)],
                              sem_s0).wait()
        pltpu.make_async_copy(sr2.at[0], rs_sh.at[pl.ds(row0, CB)],
                              sem_r0).wait()
    plsc.subcore_barrier()

    base0 = wid * PER_W
    col0 = jnp.zeros((L,), jnp.int32)
    col1 = jnp.full((L,), 1, jnp.int32)

    pltpu.async_copy(ob2.at[0], out_sh.at[hv.at[0]], sem_s0, add=True)
    pltpu.async_copy(ob2.at[1], out_sh.at[hv.at[0]], sem_s1, add=True)
    pltpu.async_copy(sr2.at[0], rs_sh.at[hv.at[0]], sem_r0, add=True)
    pltpu.async_copy(sr2.at[1], rs_sh.at[hv.at[0]], sem_r1, add=True)

    pltpu.sync_copy(h_hbm.at[pl.ds(base0, SUPE)], hflat2.at[0])
    pltpu.sync_copy(att_hbm.at[pl.ds(base0, SUPE)], aflat2.at[0])
    pltpu.sync_copy(val_hbm.at[pl.ds(base0, SUPE)], vflat2.at[0])

    def issue_g(src_b, off, bb):
        hk = hflat2.at[src_b].at[pl.ds(off, CB)]
        ak = aflat2.at[src_b].at[pl.ds(off, CB)]
        vk = vflat2.at[src_b].at[pl.ds(off, CB)]
        pltpu.async_copy(p_hbm.at[hk], ph2.at[bb], sem_p[bb])
        pltpu.async_copy(p_hbm.at[ak], pa2.at[bb], sem_p[bb])
        pltpu.async_copy(a_hbm.at[ak], ar2.at[bb], sem_g[bb])
        pltpu.async_copy(v_hbm.at[vk], vr2.at[bb], sem_g[bb])

    issue_g(0, 0, 0)

    @pl.loop(0, NSUPER)
    def _(s):
        sb = lax.rem(s, 2)

        @pl.when(s < NSUPER - 1)
        def _():
            nbase = base0 + (s + 1) * SUPE
            pltpu.async_copy(h_hbm.at[pl.ds(nbase, SUPE)],
                             hflat2.at[1 - sb], sem_i)
            pltpu.async_copy(att_hbm.at[pl.ds(nbase, SUPE)],
                             aflat2.at[1 - sb], sem_i)
            pltpu.async_copy(val_hbm.at[pl.ds(nbase, SUPE)],
                             vflat2.at[1 - sb], sem_i)

        for k in range(HPS):
            b = k & 1
            off = k * CB

            if k < HPS - 1:
                issue_g(sb, (k + 1) * CB, 1 - b)
            else:
                @pl.when(s < NSUPER - 1)
                def _():
                    for _ in range(3):
                        pltpu.make_async_copy(
                            h_hbm.at[pl.ds(base0, SUPE)], hflat2.at[0],
                            sem_i).wait()
                    issue_g(1 - sb, 0, 0)

            hv[k, pl.ds(0, L)] = hflat2[sb, pl.ds(off, L)]
            hv[k, pl.ds(L, L)] = hflat2[sb, pl.ds(off + L, L)]
            hv[k, pl.ds(CB - L, L)] = hflat2[sb, pl.ds(off + CB - L, L)]

            pltpu.make_async_copy(p_hbm.at[hflat2.at[sb].at[pl.ds(off, CB)]],
                                  ph2.at[b], sem_p[b]).wait()
            pltpu.make_async_copy(p_hbm.at[hflat2.at[sb].at[pl.ds(off, CB)]],
                                  pa2.at[b], sem_p[b]).wait()
            pltpu.make_async_copy(sr2.at[b], rs_sh.at[hv.at[k]],
                                  sem_r[b]).wait()

            for g in range(3):
                idx = jnp.minimum(lax.iota(jnp.int32, L) + g * L, CB - 1)
                x = (plsc.load_gather(ph2.at[b], [idx, col0])
                     + plsc.load_gather(pa2.at[b], [idx, col1]))
                scv = jnp.exp(jnp.maximum(x, 0.2 * x))
                if g < 2:
                    plsc.store_scatter(sr2.at[b], [idx, col0], scv)
                else:
                    plsc.store_scatter(sr2.at[b], [idx, col0], scv,
                                       mask=lax.iota(jnp.int32, L) < 8)

            pltpu.make_async_copy(a_hbm.at[hflat2.at[sb].at[pl.ds(off, CB)]],
                                  ar2.at[b], sem_g[b]).wait()
            pltpu.make_async_copy(a_hbm.at[hflat2.at[sb].at[pl.ds(off, CB)]],
                                  vr2.at[b], sem_g[b]).wait()
            pltpu.make_async_copy(ob2.at[b], out_sh.at[hv.at[k]],
                                  sem_s[b]).wait()

            @plsc.parallel_loop(0, CB, 1, unroll=4)
            def _(e):
                sbc = plsc.load_gather(
                    sr2.at[b], [jnp.full((L,), e, jnp.int32), col0])
                for q in range(K // L):
                    sl = pl.ds(q * L, L)
                    ob2[b, e, sl] = (ar2[b, e, sl] + vr2[b, e, sl]) * sbc

            pltpu.async_copy(ob2.at[b], out_sh.at[hv.at[k]], sem_s[b],
                             add=True)
            pltpu.async_copy(sr2.at[b], rs_sh.at[hv.at[k]], sem_r[b],
                             add=True)

    pltpu.make_async_copy(ob2.at[0], out_sh.at[hv.at[0]], sem_s0).wait()
    pltpu.make_async_copy(ob2.at[1], out_sh.at[hv.at[0]], sem_s1).wait()
    pltpu.make_async_copy(sr2.at[0], rs_sh.at[hv.at[0]], sem_r0).wait()
    pltpu.make_async_copy(sr2.at[1], rs_sh.at[hv.at[0]], sem_r1).wait()

    plsc.subcore_barrier()

    pltpu.sync_copy(out_sh.at[pl.ds(row0, RPS)],
                    out_hbm.at[cid].at[pl.ds(row0, RPS)])
    pltpu.sync_copy(rs_sh.at[pl.ds(row0, RPS)],
                    rs_hbm.at[cid].at[pl.ds(row0, RPS)])


def _sc_aggregate(h, att, val, a_tab, v_tab, p_tab):
    mesh = plsc.VectorSubcoreMesh(core_axis_name="c", subcore_axis_name="s")
    cp = pltpu.CompilerParams(needs_layout_passes=False,
                              use_tc_tiling_on_sc=False)
    kern = pl.kernel(
        _sc_body,
        compiler_params=cp,
        out_type=(
            jax.ShapeDtypeStruct((NC, NP, K), jnp.float32),
            jax.ShapeDtypeStruct((NC, NP, L), jnp.float32),
        ),
        mesh=mesh,
        scratch_types=[
            pltpu.VMEM((2, SUPE), jnp.int32),
            pltpu.VMEM((2, SUPE), jnp.int32),
            pltpu.VMEM((2, SUPE), jnp.int32),
            pltpu.VMEM((HPS, CB), jnp.int32),
            pltpu.VMEM((2, CB, K), jnp.float32),
            pltpu.VMEM((2, CB, K), jnp.float32),
            pltpu.VMEM((2, CB, K), jnp.float32),
            pltpu.VMEM((2, CB, L), jnp.float32),
            pltpu.VMEM((2, CB, L), jnp.float32),
            pltpu.VMEM((2, CB, L), jnp.float32),
            pltpu.VMEM_SHARED((NP, K), jnp.float32),
            pltpu.VMEM_SHARED((NP, L), jnp.float32),
            pltpu.SemaphoreType.DMA,
            pltpu.SemaphoreType.DMA,
            pltpu.SemaphoreType.DMA,
            pltpu.SemaphoreType.DMA,
            pltpu.SemaphoreType.DMA,
            pltpu.SemaphoreType.DMA,
            pltpu.SemaphoreType.DMA,
            pltpu.SemaphoreType.DMA,
            pltpu.SemaphoreType.DMA,
        ],
    )
    return kern(h, att, val, a_tab, v_tab, p_tab)


def _combine_body(po_ref, prs_ref, ent_ref, out_ref):
    tot = po_ref[0, 0:N] + po_ref[1, 0:N]
    rs = prs_ref[0, 0:N, 0] + prs_ref[1, 0:N, 0]
    rs = jnp.where(rs == 0.0, 1.0, rs)
    t = tot / rs[:, None] + ent_ref[...]
    out_ref[...] = jnp.where(t > 0.0, t, jnp.exp(jnp.minimum(t, 0.0)) - 1.0)


def _combine(out_pair, rs_pair, ent_feats):
    return pl.pallas_call(
        _combine_body,
        out_shape=jax.ShapeDtypeStruct((N, K), jnp.float32),
    )(out_pair, rs_pair, ent_feats)


def kernel(attribute_triples, att_feats, val_feats, ent_feats, a_w, a_b, W):
    h = attribute_triples[:, 0]
    val = attribute_triples[:, 1]
    att = attribute_triples[:, 2]
    a_tab, v_tab, p_tab = _project(att_feats, val_feats, ent_feats,
                                   a_w, a_b, W)
    out_pair, rs_pair = _sc_aggregate(h, att, val, a_tab, v_tab, p_tab)
    return _combine(out_pair, rs_pair, ent_feats)


# speedup vs baseline: 1.4514x; 1.0076x over previous
import jax
import jax.numpy as jnp
from jax import lax
from jax.experimental import pallas as pl
from jax.experimental.pallas import tpu as pltpu
from jax.experimental.pallas import tpu_sc as plsc

N = 10000
E = 320000
K = 128
NC = 2
NS = 16
NW = NC * NS
L = 16
CB = 40
HPS = 10
SUPE = CB * HPS
PER_W = E // NW
NSUPER = PER_W // SUPE
NP = 10240
RPS = NP // NS


def _proj_body(att_ref, val_ref, ent_ref, aw_ref, ab_ref, w_ref,
               a_ref, v_ref, p_ref):
    w1 = w_ref[0:K, :]
    w2 = w_ref[K:2 * K, :]
    a_ref[...] = jnp.dot(att_ref[...], w1, preferred_element_type=jnp.float32)
    v_ref[...] = jnp.dot(val_ref[...], w2, preferred_element_type=jnp.float32)
    aw1 = aw_ref[0, 0:K]
    aw2 = aw_ref[0, K:2 * K]
    se = jnp.sum(ent_ref[...] * aw1[None, :], axis=1, keepdims=True)
    se = se + ab_ref[0, 0]
    sa = jnp.sum(att_ref[...] * aw2[None, :], axis=1, keepdims=True)
    p_ref[...] = jnp.concatenate(
        (se, sa, jnp.zeros((N, L - 2), jnp.float32)), axis=1)


def _project(att_feats, val_feats, ent_feats, a_w, a_b, w):
    return pl.pallas_call(
        _proj_body,
        out_shape=(
            jax.ShapeDtypeStruct((N, K), jnp.float32),
            jax.ShapeDtypeStruct((N, K), jnp.float32),
            jax.ShapeDtypeStruct((N, L), jnp.float32),
        ),
    )(att_feats, val_feats, ent_feats, a_w, a_b.reshape(1, 1), w)


def _sc_body(h_hbm, att_hbm, val_hbm, a_hbm, v_hbm, p_hbm,
             out_hbm, rs_hbm,
             hflat2, aflat2, vflat2, hv, ar2, vr2, ob2, sr2, ph2, pa2,
             out_sh, rs_sh,
             sem_g0, sem_g1, sem_p0, sem_p1,
             sem_s0, sem_s1, sem_r0, sem_r1, sem_i):
    cid = lax.axis_index("c")
    sid = lax.axis_index("s")
    wid = sid * NC + cid

    sem_g = (sem_g0, sem_g1)
    sem_p = (sem_p0, sem_p1)
    sem_s = (sem_s0, sem_s1)
    sem_r = (sem_r0, sem_r1)

    zeros_f = jnp.zeros((L,), jnp.float32)
    zeros_i = jnp.zeros((L,), jnp.int32)

    @pl.loop(0, CB)
    def _(r):
        for bb in range(2):
            for q in range(K // L):
                ob2[bb, r, pl.ds(q * L, L)] = zeros_f
            sr2[bb, r, :] = zeros_f

    for kk in range(HPS):
        hv[kk, pl.ds(0, L)] = zeros_i
        hv[kk, pl.ds(L, L)] = zeros_i
        hv[kk, pl.ds(CB - L, L)] = zeros_i

    row0 = sid * RPS
    for i in range(RPS // CB):
        pltpu.async_copy(ob2.at[0], out_sh.at[pl.ds(row0 + i * CB, CB)],
                         sem_s0)
        pltpu.async_copy(sr2.at[0], rs_sh.at[pl.ds(row0 + i * CB, CB)],
                         sem_r0)
    for i in range(RPS // CB):
        pltpu.make_async_copy(ob2.at[0], out_sh.at[pl.ds(row0, CB)],
                              sem_s0).wait()
        pltpu.make_async_copy(sr2.at[0], rs_sh.at[pl.ds(row0, CB)],
                              sem_r0).wait()
    plsc.subcore_barrier()

    base0 = wid * PER_W
    col0 = jnp.zeros((L,), jnp.int32)
    col1 = jnp.full((L,), 1, jnp.int32)

    pltpu.async_copy(ob2.at[0], out_sh.at[hv.at[0]], sem_s0, add=True)
    pltpu.async_copy(ob2.at[1], out_sh.at[hv.at[0]], sem_s1, add=True)
    pltpu.async_copy(sr2.at[0], rs_sh.at[hv.at[0]], sem_r0, add=True)
    pltpu.async_copy(sr2.at[1], rs_sh.at[hv.at[0]], sem_r1, add=True)

    pltpu.sync_copy(h_hbm.at[pl.ds(base0, SUPE)], hflat2.at[0])
    pltpu.sync_copy(att_hbm.at[pl.ds(base0, SUPE)], aflat2.at[0])
    pltpu.sync_copy(val_hbm.at[pl.ds(base0, SUPE)], vflat2.at[0])

    def issue_g(src_b, off, bb):
        hk = hflat2.at[src_b].at[pl.ds(off, CB)]
        ak = aflat2.at[src_b].at[pl.ds(off, CB)]
        vk = vflat2.at[src_b].at[pl.ds(off, CB)]
        pltpu.async_copy(p_hbm.at[hk], ph2.at[bb], sem_p[bb])
        pltpu.async_copy(p_hbm.at[ak], pa2.at[bb], sem_p[bb])
        pltpu.async_copy(a_hbm.at[ak], ar2.at[bb], sem_g[bb])
        pltpu.async_copy(v_hbm.at[vk], vr2.at[bb], sem_g[bb])

    issue_g(0, 0, 0)

    @pl.loop(0, NSUPER)
    def _(s):
        sb = lax.rem(s, 2)

        @pl.when(s < NSUPER - 1)
        def _():
            nbase = base0 + (s + 1) * SUPE
            pltpu.async_copy(h_hbm.at[pl.ds(nbase, SUPE)],
                             hflat2.at[1 - sb], sem_i)
            pltpu.async_copy(att_hbm.at[pl.ds(nbase, SUPE)],
                             aflat2.at[1 - sb], sem_i)
            pltpu.async_copy(val_hbm.at[pl.ds(nbase, SUPE)],
                             vflat2.at[1 - sb], sem_i)

        for k in range(HPS):
            b = k & 1
            off = k * CB

            if k < HPS - 1:
                issue_g(sb, (k + 1) * CB, 1 - b)
            else:
                @pl.when(s < NSUPER - 1)
                def _():
                    for _ in range(3):
                        pltpu.make_async_copy(
                            h_hbm.at[pl.ds(base0, SUPE)], hflat2.at[0],
                            sem_i).wait()
                    issue_g(1 - sb, 0, 0)

            hv[k, pl.ds(0, L)] = hflat2[sb, pl.ds(off, L)]
            hv[k, pl.ds(L, L)] = hflat2[sb, pl.ds(off + L, L)]
            hv[k, pl.ds(CB - L, L)] = hflat2[sb, pl.ds(off + CB - L, L)]

            pltpu.make_async_copy(p_hbm.at[hflat2.at[sb].at[pl.ds(off, CB)]],
                                  ph2.at[b], sem_p[b]).wait()
            pltpu.make_async_copy(p_hbm.at[hflat2.at[sb].at[pl.ds(off, CB)]],
                                  pa2.at[b], sem_p[b]).wait()
            pltpu.make_async_copy(sr2.at[b], rs_sh.at[hv.at[k]],
                                  sem_r[b]).wait()

            for g in range(3):
                idx = jnp.minimum(lax.iota(jnp.int32, L) + g * L, CB - 1)
                x = (plsc.load_gather(ph2.at[b], [idx, col0])
                     + plsc.load_gather(pa2.at[b], [idx, col1]))
                scv = jnp.exp(jnp.maximum(x, 0.2 * x))
                if g < 2:
                    plsc.store_scatter(sr2.at[b], [idx, col0], scv)
                else:
                    plsc.store_scatter(sr2.at[b], [idx, col0], scv,
                                       mask=lax.iota(jnp.int32, L) < 8)

            pltpu.make_async_copy(a_hbm.at[hflat2.at[sb].at[pl.ds(off, CB)]],
                                  ar2.at[b], sem_g[b]).wait()
            pltpu.make_async_copy(a_hbm.at[hflat2.at[sb].at[pl.ds(off, CB)]],
                                  vr2.at[b], sem_g[b]).wait()
            pltpu.make_async_copy(ob2.at[b], out_sh.at[hv.at[k]],
                                  sem_s[b]).wait()

            @plsc.parallel_loop(0, CB, 1, unroll=2)
            def _(e):
                sbc = plsc.load_gather(
                    sr2.at[b], [jnp.full((L,), e, jnp.int32), col0])
                for q in range(K // L):
                    sl = pl.ds(q * L, L)
                    ob2[b, e, sl] = (ar2[b, e, sl] + vr2[b, e, sl]) * sbc

            pltpu.async_copy(ob2.at[b], out_sh.at[hv.at[k]], sem_s[b],
                             add=True)
            pltpu.async_copy(sr2.at[b], rs_sh.at[hv.at[k]], sem_r[b],
                             add=True)

    pltpu.make_async_copy(ob2.at[0], out_sh.at[hv.at[0]], sem_s0).wait()
    pltpu.make_async_copy(ob2.at[1], out_sh.at[hv.at[0]], sem_s1).wait()
    pltpu.make_async_copy(sr2.at[0], rs_sh.at[hv.at[0]], sem_r0).wait()
    pltpu.make_async_copy(sr2.at[1], rs_sh.at[hv.at[0]], sem_r1).wait()

    plsc.subcore_barrier()

    pltpu.sync_copy(out_sh.at[pl.ds(row0, RPS)],
                    out_hbm.at[cid].at[pl.ds(row0, RPS)])
    pltpu.sync_copy(rs_sh.at[pl.ds(row0, RPS)],
                    rs_hbm.at[cid].at[pl.ds(row0, RPS)])


def _sc_aggregate(h, att, val, a_tab, v_tab, p_tab):
    mesh = plsc.VectorSubcoreMesh(core_axis_name="c", subcore_axis_name="s")
    cp = pltpu.CompilerParams(needs_layout_passes=False,
                              use_tc_tiling_on_sc=False)
    kern = pl.kernel(
        _sc_body,
        compiler_params=cp,
        out_type=(
            jax.ShapeDtypeStruct((NC, NP, K), jnp.float32),
            jax.ShapeDtypeStruct((NC, NP, L), jnp.float32),
        ),
        mesh=mesh,
        scratch_types=[
            pltpu.VMEM((2, SUPE), jnp.int32),
            pltpu.VMEM((2, SUPE), jnp.int32),
            pltpu.VMEM((2, SUPE), jnp.int32),
            pltpu.VMEM((HPS, CB), jnp.int32),
            pltpu.VMEM((2, CB, K), jnp.float32),
            pltpu.VMEM((2, CB, K), jnp.float32),
            pltpu.VMEM((2, CB, K), jnp.float32),
            pltpu.VMEM((2, CB, L), jnp.float32),
            pltpu.VMEM((2, CB, L), jnp.float32),
            pltpu.VMEM((2, CB, L), jnp.float32),
            pltpu.VMEM_SHARED((NP, K), jnp.float32),
            pltpu.VMEM_SHARED((NP, L), jnp.float32),
            pltpu.SemaphoreType.DMA,
            pltpu.SemaphoreType.DMA,
            pltpu.SemaphoreType.DMA,
            pltpu.SemaphoreType.DMA,
            pltpu.SemaphoreType.DMA,
            pltpu.SemaphoreType.DMA,
            pltpu.SemaphoreType.DMA,
            pltpu.SemaphoreType.DMA,
            pltpu.SemaphoreType.DMA,
        ],
    )
    return kern(h, att, val, a_tab, v_tab, p_tab)


def _combine_body(po_ref, prs_ref, ent_ref, out_ref):
    tot = po_ref[0, 0:N] + po_ref[1, 0:N]
    rs = prs_ref[0, 0:N, 0] + prs_ref[1, 0:N, 0]
    rs = jnp.where(rs == 0.0, 1.0, rs)
    t = tot / rs[:, None] + ent_ref[...]
    out_ref[...] = jnp.where(t > 0.0, t, jnp.exp(jnp.minimum(t, 0.0)) - 1.0)


def _combine(out_pair, rs_pair, ent_feats):
    return pl.pallas_call(
        _combine_body,
        out_shape=jax.ShapeDtypeStruct((N, K), jnp.float32),
    )(out_pair, rs_pair, ent_feats)


def kernel(attribute_triples, att_feats, val_feats, ent_feats, a_w, a_b, W):
    h = attribute_triples[:, 0]
    val = attribute_triples[:, 1]
    att = attribute_triples[:, 2]
    a_tab, v_tab, p_tab = _project(att_feats, val_feats, ent_feats,
                                   a_w, a_b, W)
    out_pair, rs_pair = _sc_aggregate(h, att, val, a_tab, v_tab, p_tab)
    return _combine(out_pair, rs_pair, ent_feats)
